# baseline (speedup 1.0000x reference)
import dataclasses
import functools

import jax
import jax.numpy as jnp
from jax import lax
from jax.experimental import pallas as pl
from jax.experimental.pallas import tpu as pltpu
from jax.experimental.pallas import tpu_sc as plsc

NC, NS, L = 2, 16, 16
NW = NC * NS
N_NODES = 10000
N_PAD = 10240
F = 64
NE = 320000
NE_PAD = 327680
EPT = NE_PAD // NW
BLK = 1024
NBLK = EPT // BLK
CH = 128
NCHT = EPT // CH
NB = 4
CH0 = 64
CH1 = 96
CHMX = max(CH0, CH1)
ZR = CH
RPT = N_PAD // NS


@functools.cache
def _mesh():
    return plsc.VectorSubcoreMesh(
        core_axis_name="c", subcore_axis_name="s",
        num_cores=NC, num_subcores=NS,
    )


def _wid():
    return lax.axis_index("s") * NC + lax.axis_index("c")


@functools.cache
def _sc_params():
    cp = pltpu.CompilerParams()
    fields = pltpu.CompilerParams.__dataclass_fields__
    if "needs_layout_passes" in fields:
        cp = dataclasses.replace(cp, needs_layout_passes=False)
    if "use_tc_tiling_on_sc" in fields:
        cp = dataclasses.replace(cp, use_tc_tiling_on_sc=False)
    return cp


def _proj_body(x_ref, w_ref, o_ref):
    o_ref[...] = jnp.dot(x_ref[...], w_ref[...],
                         preferred_element_type=jnp.float32)


def _project(x_pad, wt):
    return pl.pallas_call(
        _proj_body,
        grid=(N_PAD // 1024,),
        in_specs=[
            pl.BlockSpec((1024, 128), lambda i: (i, 0)),
            pl.BlockSpec((128, F), lambda i: (0, 0)),
        ],
        out_specs=pl.BlockSpec((1024, F), lambda i: (i, 0)),
        out_shape=jax.ShapeDtypeStruct((N_PAD, F), jnp.float32),
    )(x_pad, wt)


def _deg_body(col_hbm, ew_hbm, degp_hbm, colb, ewb, dacc):
    wid = _wid()
    zero = jnp.zeros((L,), jnp.float32)

    @pl.loop(0, N_PAD // L)
    def _(i):
        dacc[pl.ds(i * L, L)] = zero

    @pl.loop(0, NBLK)
    def _(bk):
        pltpu.sync_copy(col_hbm.at[wid, bk], colb)
        pltpu.sync_copy(ew_hbm.at[wid, bk], ewb)

        @pl.loop(0, BLK // L)
        def _(g):
            idx = colb[pl.ds(g * L, L)]
            val = ewb[pl.ds(g * L, L)]
            plsc.addupdate_scatter(dacc, [idx], val)

    pltpu.sync_copy(dacc, degp_hbm.at[wid])


def _degrees(col2, ew2):
    k = pl.kernel(
        _deg_body,
        out_type=jax.ShapeDtypeStruct((NW, N_PAD), jnp.float32),
        mesh=_mesh(),
        compiler_params=_sc_params(),
        scratch_types=[
            pltpu.VMEM((BLK,), jnp.int32),
            pltpu.VMEM((BLK,), jnp.float32),
            pltpu.VMEM((N_PAD,), jnp.float32),
        ],
    )
    return k(col2, ew2)


def _prep_body(p_ref, h_ref, dd_ref, g_ref):
    deg = jnp.sum(p_ref[...], axis=0, keepdims=True) + 1.0
    dinv = jnp.where(deg > 0, lax.rsqrt(deg), 0.0)
    dd_ref[...] = jnp.concatenate([dinv, dinv * dinv], axis=0)
    g_ref[...] = dinv.reshape(N_PAD, 1) * h_ref[...]


def _prep(degp, h0):
    return pl.pallas_call(
        _prep_body,
        out_shape=(
            jax.ShapeDtypeStruct((2, N_PAD), jnp.float32),
            jax.ShapeDtypeStruct((N_PAD, F), jnp.float32),
        ),
    )(degp, h0)


def _prop_body(h_hbm, row_hbm, col_hbm, ew_hbm, out_hbm,
               rowb, colb, nb, b0, b1, b2, b3,
               acc, g0, g1, g2, g3, s0, s1, s2, s3):
    bufs = [b0, b1, b2, b3]
    gsems = [g0, g1, g2, g3]
    ssems = [s0, s1, s2, s3]
    cid = lax.axis_index("c")
    sid = lax.axis_index("s")
    wid = sid * NC + cid

    @pl.loop(0, ZR)
    def _(i):
        for f in range(F // L):
            b0[i, pl.ds(f * L, L)] = jnp.zeros((L,), jnp.float32)

    @pl.loop(0, RPT // ZR)
    def _(i):
        pltpu.sync_copy(b0, acc.at[pl.ds(sid * RPT + i * ZR, ZR)])

    base = jnp.where(cid == 0, sid * CH0, NS * CH0 + sid * CH1)
    ngroups = jnp.where(cid == 0, CH0 // NB, CH1 // NB)
    pltpu.sync_copy(row_hbm.at[pl.ds(base, CHMX)], rowb)
    pltpu.sync_copy(col_hbm.at[pl.ds(base, CHMX)], colb)
    pltpu.sync_copy(ew_hbm.at[pl.ds(base * CH, CHMX * CH)], nb)

    plsc.subcore_barrier()

    @pl.loop(0, ngroups)
    def _(i):
        gds = []
        for k in range(NB):
            ch = i * NB + k
            gds.append(pltpu.async_copy(
                h_hbm.at[rowb.at[ch]], bufs[k], gsems[k]))

        sds = []
        for k in range(NB):
            ch = i * NB + k
            gds[k].wait()
            base = ch * CH

            @plsc.parallel_loop(0, CH, unroll=2)
            def _(e, _k=k):
                nv = plsc.load_gather(
                    nb, [jnp.full((L,), base + e, jnp.int32)])
                for f in range(F // L):
                    sl = (e, pl.ds(f * L, L))
                    bufs[_k][sl] = bufs[_k][sl] * nv

            sds.append(pltpu.async_copy(
                bufs[k], acc.at[colb.at[ch]], ssems[k], add=True))

        for k in range(NB):
            sds[k].wait()

    plsc.subcore_barrier()

    @pl.loop(0, RPT // ZR)
    def _(i):
        r = sid * RPT + i * ZR
        pltpu.sync_copy(acc.at[pl.ds(r, ZR)], out_hbm.at[cid, pl.ds(r, ZR)])


def _propagate(g, row3, col3, ew1):
    k = pl.kernel(
        _prop_body,
        out_type=jax.ShapeDtypeStruct((NC, N_PAD, F), jnp.float32),
        mesh=_mesh(),
        compiler_params=_sc_params(),
        scratch_types=(
            [
                pltpu.VMEM((CHMX, CH), jnp.int32),
                pltpu.VMEM((CHMX, CH), jnp.int32),
                pltpu.VMEM((CHMX * CH,), jnp.float32),
            ]
            + [pltpu.VMEM((CH, F), jnp.float32) for _ in range(NB)]
            + [pltpu.VMEM_SHARED((N_PAD, F), jnp.float32)]
            + [pltpu.SemaphoreType.DMA for _ in range(2 * NB)]
        ),
    )
    return k(g, row3, col3, ew1)


def _comb_body(p_ref, h_ref, s_ref, b_ref, o_ref):
    o_ref[...] = s_ref[...] * (p_ref[0] + p_ref[1] + h_ref[...]) + b_ref[...]


def _combine(p, h, scol, bias_row):
    return pl.pallas_call(
        _comb_body,
        out_shape=jax.ShapeDtypeStruct((N_PAD, F), jnp.float32),
    )(p, h, scol, bias_row)


@jax.jit
def kernel(x, edge_index, edge_weight, W, b):
    row = edge_index[0].astype(jnp.int32)
    col = edge_index[1].astype(jnp.int32)
    ew = edge_weight.astype(jnp.float32)

    pad_e = NE_PAD - NE
    row_p = jnp.concatenate([row, jnp.zeros((pad_e,), jnp.int32)])
    col_p = jnp.concatenate([col, jnp.zeros((pad_e,), jnp.int32)])
    ew_p = jnp.concatenate([ew, jnp.zeros((pad_e,), jnp.float32)])

    col2 = col_p.reshape(NW, NBLK, BLK)
    ew2 = ew_p.reshape(NW, NBLK, BLK)
    row3 = row_p.reshape(NE_PAD // CH, CH)
    col3 = col_p.reshape(NE_PAD // CH, CH)
    ew1 = ew_p

    x_pad = jnp.concatenate(
        [x, jnp.zeros((N_PAD - N_NODES, x.shape[1]), jnp.float32)])
    wt = W.T.astype(jnp.float32)

    h0 = _project(x_pad, wt)
    degp = _degrees(col2, ew2)
    dd, g0 = _prep(degp, h0)
    dcol = dd[0][:, None]
    d2col = dd[1][:, None]

    zero_b = jnp.zeros((1, F), jnp.float32)
    p1 = _propagate(g0, row3, col3, ew1)
    g1 = _combine(p1, g0, d2col, zero_b)
    p2 = _propagate(g1, row3, col3, ew1)
    out = _combine(p2, g1, dcol, b.reshape(1, F).astype(jnp.float32))

    return out[:N_NODES]


# speedup vs baseline: 1.1955x; 1.1955x over previous
import dataclasses
import functools

import jax
import jax.numpy as jnp
from jax import lax
from jax.experimental import pallas as pl
from jax.experimental.pallas import tpu as pltpu
from jax.experimental.pallas import tpu_sc as plsc

NC, NS, L = 2, 16, 16
NW = NC * NS
N_NODES = 10000
N_PAD = 10240
F = 64
NE = 320000
NE_PAD = 327680
EPT = NE_PAD // NW
BLK = 1024
NBLK = EPT // BLK
CH = 128
NCHT = EPT // CH
NB = 4
CH0 = 144
CH1 = 16
CHMX = max(CH0, CH1)
ZR = CH
RPT = N_PAD // NS


@functools.cache
def _mesh():
    return plsc.VectorSubcoreMesh(
        core_axis_name="c", subcore_axis_name="s",
        num_cores=NC, num_subcores=NS,
    )


def _wid():
    return lax.axis_index("s") * NC + lax.axis_index("c")


@functools.cache
def _sc_params():
    cp = pltpu.CompilerParams()
    fields = pltpu.CompilerParams.__dataclass_fields__
    if "needs_layout_passes" in fields:
        cp = dataclasses.replace(cp, needs_layout_passes=False)
    if "use_tc_tiling_on_sc" in fields:
        cp = dataclasses.replace(cp, use_tc_tiling_on_sc=False)
    return cp


def _proj_body(x_ref, w_ref, o_ref):
    o_ref[...] = jnp.dot(x_ref[...], w_ref[...],
                         preferred_element_type=jnp.float32)


def _project(x_pad, wt):
    return pl.pallas_call(
        _proj_body,
        grid=(N_PAD // 1024,),
        in_specs=[
            pl.BlockSpec((1024, 128), lambda i: (i, 0)),
            pl.BlockSpec((128, F), lambda i: (0, 0)),
        ],
        out_specs=pl.BlockSpec((1024, F), lambda i: (i, 0)),
        out_shape=jax.ShapeDtypeStruct((N_PAD, F), jnp.float32),
    )(x_pad, wt)


def _deg_body(col_hbm, ew_hbm, degp_hbm, colb, ewb, dacc):
    wid = _wid()
    zero = jnp.zeros((L,), jnp.float32)

    @pl.loop(0, N_PAD // L)
    def _(i):
        dacc[pl.ds(i * L, L)] = zero

    @pl.loop(0, NBLK)
    def _(bk):
        pltpu.sync_copy(col_hbm.at[wid, bk], colb)
        pltpu.sync_copy(ew_hbm.at[wid, bk], ewb)

        @pl.loop(0, BLK // L)
        def _(g):
            idx = colb[pl.ds(g * L, L)]
            val = ewb[pl.ds(g * L, L)]
            plsc.addupdate_scatter(dacc, [idx], val)

    pltpu.sync_copy(dacc, degp_hbm.at[wid])


def _degrees(col2, ew2):
    k = pl.kernel(
        _deg_body,
        out_type=jax.ShapeDtypeStruct((NW, N_PAD), jnp.float32),
        mesh=_mesh(),
        compiler_params=_sc_params(),
        scratch_types=[
            pltpu.VMEM((BLK,), jnp.int32),
            pltpu.VMEM((BLK,), jnp.float32),
            pltpu.VMEM((N_PAD,), jnp.float32),
        ],
    )
    return k(col2, ew2)


def _prep_body(p_ref, h_ref, dd_ref, g_ref):
    deg = jnp.sum(p_ref[...], axis=0, keepdims=True) + 1.0
    dinv = jnp.where(deg > 0, lax.rsqrt(deg), 0.0)
    dd_ref[...] = jnp.concatenate([dinv, dinv * dinv], axis=0)
    g_ref[...] = dinv.reshape(N_PAD, 1) * h_ref[...]


def _prep(degp, h0):
    return pl.pallas_call(
        _prep_body,
        out_shape=(
            jax.ShapeDtypeStruct((2, N_PAD), jnp.float32),
            jax.ShapeDtypeStruct((N_PAD, F), jnp.float32),
        ),
    )(degp, h0)


def _prop_body(h_hbm, row_hbm, col_hbm, ew_hbm, out_hbm,
               rowb, colb, nb, b0, b1, b2, b3,
               acc, g0, g1, g2, g3, s0, s1, s2, s3):
    bufs = [b0, b1, b2, b3]
    gsems = [g0, g1, g2, g3]
    ssems = [s0, s1, s2, s3]
    cid = lax.axis_index("c")
    sid = lax.axis_index("s")
    wid = sid * NC + cid

    @pl.loop(0, ZR)
    def _(i):
        for f in range(F // L):
            b0[i, pl.ds(f * L, L)] = jnp.zeros((L,), jnp.float32)

    @pl.loop(0, RPT // ZR)
    def _(i):
        pltpu.sync_copy(b0, acc.at[pl.ds(sid * RPT + i * ZR, ZR)])

    base = jnp.where(cid == 0, sid * CH0, NS * CH0 + sid * CH1)
    ngroups = jnp.where(cid == 0, CH0 // NB, CH1 // NB)

    @pl.when(cid == 0)
    def _():
        pltpu.sync_copy(row_hbm.at[pl.ds(base, CH0)], rowb.at[pl.ds(0, CH0)])
        pltpu.sync_copy(col_hbm.at[pl.ds(base, CH0)], colb.at[pl.ds(0, CH0)])
        pltpu.sync_copy(ew_hbm.at[pl.ds(base * CH, CH0 * CH)],
                        nb.at[pl.ds(0, CH0 * CH)])

    @pl.when(cid == 1)
    def _():
        pltpu.sync_copy(row_hbm.at[pl.ds(base, CH1)], rowb.at[pl.ds(0, CH1)])
        pltpu.sync_copy(col_hbm.at[pl.ds(base, CH1)], colb.at[pl.ds(0, CH1)])
        pltpu.sync_copy(ew_hbm.at[pl.ds(base * CH, CH1 * CH)],
                        nb.at[pl.ds(0, CH1 * CH)])

    plsc.subcore_barrier()

    @pl.loop(0, ngroups)
    def _(i):
        gds = []
        for k in range(NB):
            ch = i * NB + k
            gds.append(pltpu.async_copy(
                h_hbm.at[rowb.at[ch]], bufs[k], gsems[k]))

        sds = []
        for k in range(NB):
            ch = i * NB + k
            gds[k].wait()
            base = ch * CH

            @plsc.parallel_loop(0, CH, unroll=2)
            def _(e, _k=k):
                nv = plsc.load_gather(
                    nb, [jnp.full((L,), base + e, jnp.int32)])
                for f in range(F // L):
                    sl = (e, pl.ds(f * L, L))
                    bufs[_k][sl] = bufs[_k][sl] * nv

            sds.append(pltpu.async_copy(
                bufs[k], acc.at[colb.at[ch]], ssems[k], add=True))

        for k in range(NB):
            sds[k].wait()

    plsc.subcore_barrier()

    @pl.loop(0, RPT // ZR)
    def _(i):
        r = sid * RPT + i * ZR
        pltpu.sync_copy(acc.at[pl.ds(r, ZR)], out_hbm.at[cid, pl.ds(r, ZR)])


def _propagate(g, row3, col3, ew1):
    k = pl.kernel(
        _prop_body,
        out_type=jax.ShapeDtypeStruct((NC, N_PAD, F), jnp.float32),
        mesh=_mesh(),
        compiler_params=_sc_params(),
        scratch_types=(
            [
                pltpu.VMEM((CHMX, CH), jnp.int32),
                pltpu.VMEM((CHMX, CH), jnp.int32),
                pltpu.VMEM((CHMX * CH,), jnp.float32),
            ]
            + [pltpu.VMEM((CH, F), jnp.float32) for _ in range(NB)]
            + [pltpu.VMEM_SHARED((N_PAD, F), jnp.float32)]
            + [pltpu.SemaphoreType.DMA for _ in range(2 * NB)]
        ),
    )
    return k(g, row3, col3, ew1)


def _comb_body(p_ref, h_ref, s_ref, b_ref, o_ref):
    o_ref[...] = s_ref[...] * (p_ref[0] + p_ref[1] + h_ref[...]) + b_ref[...]


def _combine(p, h, scol, bias_row):
    return pl.pallas_call(
        _comb_body,
        out_shape=jax.ShapeDtypeStruct((N_PAD, F), jnp.float32),
    )(p, h, scol, bias_row)


@jax.jit
def kernel(x, edge_index, edge_weight, W, b):
    row = edge_index[0].astype(jnp.int32)
    col = edge_index[1].astype(jnp.int32)
    ew = edge_weight.astype(jnp.float32)

    pad_e = NE_PAD - NE
    row_p = jnp.concatenate([row, jnp.zeros((pad_e,), jnp.int32)])
    col_p = jnp.concatenate([col, jnp.zeros((pad_e,), jnp.int32)])
    ew_p = jnp.concatenate([ew, jnp.zeros((pad_e,), jnp.float32)])

    col2 = col_p.reshape(NW, NBLK, BLK)
    ew2 = ew_p.reshape(NW, NBLK, BLK)
    row3 = row_p.reshape(NE_PAD // CH, CH)
    col3 = col_p.reshape(NE_PAD // CH, CH)
    ew1 = ew_p

    x_pad = jnp.concatenate(
        [x, jnp.zeros((N_PAD - N_NODES, x.shape[1]), jnp.float32)])
    wt = W.T.astype(jnp.float32)

    h0 = _project(x_pad, wt)
    degp = _degrees(col2, ew2)
    dd, g0 = _prep(degp, h0)
    dcol = dd[0][:, None]
    d2col = dd[1][:, None]

    zero_b = jnp.zeros((1, F), jnp.float32)
    p1 = _propagate(g0, row3, col3, ew1)
    g1 = _combine(p1, g0, d2col, zero_b)
    p2 = _propagate(g1, row3, col3, ew1)
    out = _combine(p2, g1, dcol, b.reshape(1, F).astype(jnp.float32))

    return out[:N_NODES]
